# Initial kernel scaffold; baseline (speedup 1.0000x reference)
import jax
import jax.numpy as jnp
from jax.experimental import pallas as pl


def kernel(image_shape, boxes, classification, fpn0, fpn1, fpn2, fpn3, fpn4):
    raise NotImplementedError("write your pallas kernel here")


# speedup vs baseline: 16.2567x; 16.2567x over previous
import functools

import jax
import jax.numpy as jnp
from jax.experimental import pallas as pl
from jax.experimental.pallas import tpu as pltpu

_CROP = 14
_K = 500
_WMAX = 128


def _scores_kernel(cls_ref, out_ref):
    out_ref[...] = jnp.max(cls_ref[...], axis=1, keepdims=True)


def _roi_kernel(sel_ref, ys_ref, pi_ref, boxes_ref, cls_ref, table_ref, xs_ref,
                boxes_out_ref, cls_out_ref, rois_ref):
    j = pl.program_id(0)
    src = sel_ref[j]

    boxes_out_ref[...] = boxes_ref[pl.ds(src, 1), :][None]
    cls_out_ref[...] = cls_ref[pl.ds(src, 1), :][None]

    h_i = pi_ref[j, 0]
    w_i = pi_ref[j, 1]
    wp_i = pi_ref[j, 2]
    base = pi_ref[j, 3]
    hf = h_i.astype(jnp.float32)
    wf = w_i.astype(jnp.float32)

    col = jax.lax.broadcasted_iota(jnp.int32, (_CROP, _WMAX), 1).astype(jnp.float32)
    xs = xs_ref[0]
    x0f = jnp.floor(xs)
    wx = xs - x0f
    x0 = jnp.clip(x0f, 0.0, wf - 1.0)
    xp = jnp.clip(x0f + 1.0, 0.0, wf - 1.0)
    vx = ((xs >= 0.0) & (xs <= wf - 1.0)).astype(jnp.float32)
    cmat = ((col == x0).astype(jnp.float32) * (1.0 - wx)
            + (col == xp).astype(jnp.float32) * wx) * vx

    def body(iy, _):
        ysv = ys_ref[j, iy]
        y0f = jnp.floor(ysv)
        wy = ysv - y0f
        y0 = jnp.clip(y0f, 0.0, hf - 1.0).astype(jnp.int32)
        yp = jnp.clip(y0f + 1.0, 0.0, hf - 1.0).astype(jnp.int32)
        vy = ((ysv >= 0.0) & (ysv <= hf - 1.0)).astype(jnp.float32)
        s0 = pl.multiple_of(base + y0 * wp_i, 8)
        s1 = pl.multiple_of(base + yp * wp_i, 8)
        r0 = table_ref[pl.ds(s0, _WMAX), :]
        r1 = table_ref[pl.ds(s1, _WMAX), :]
        blend = (r0 * (1.0 - wy) + r1 * wy) * vy
        out_iy = jnp.dot(cmat, blend, preferred_element_type=jnp.float32)
        rois_ref[0, pl.ds(iy, 1), :, :] = out_iy[None]
        return 0

    jax.lax.fori_loop(0, _CROP, body, 0)


def kernel(image_shape, boxes, classification, fpn0, fpn1, fpn2, fpn3, fpn4):
    fpn = [fpn0[0], fpn1[0], fpn2[0], fpn3[0], fpn4[0]]
    b_all = boxes[0]
    cls_all = classification[0]
    n = b_all.shape[0]
    nc = cls_all.shape[1]
    c = fpn[0].shape[-1]
    imgf = image_shape.astype(jnp.float32)

    scores = pl.pallas_call(
        _scores_kernel,
        out_shape=jax.ShapeDtypeStruct((n, 1), jnp.float32),
    )(cls_all)[:, 0]

    k = min(_K, n)
    _, idx = jax.lax.top_k(scores, k)
    bsel = jnp.take(b_all, idx, axis=0)

    bw = bsel[:, 2] - bsel[:, 0]
    bh = bsel[:, 3] - bsel[:, 1]
    size = jnp.sqrt(bw * bh)
    levels = jnp.floor(1.0 + jnp.log2(size / 224.0 + 1e-7))
    levels = jnp.clip(levels, 0.0, 4.0).astype(jnp.int32)
    order = jnp.argsort(levels, stable=True)
    sel = jnp.take(idx, order, axis=0)
    lev = jnp.take(levels, order, axis=0)
    bs = jnp.take(bsel, order, axis=0)

    hs = [f.shape[0] for f in fpn]
    ws = [f.shape[1] for f in fpn]
    wps = [ww + (-ww) % 8 for ww in ws]
    bases = []
    acc = 0
    for hh, wp in zip(hs, wps):
        bases.append(acc)
        acc += hh * wp
    rows = acc + _WMAX
    rows += (-rows) % 8
    table = jnp.concatenate(
        [jnp.pad(f, ((0, 0), (0, wp - f.shape[1]), (0, 0))).reshape(-1, c)
         for f, wp in zip(fpn, wps)], axis=0)
    table = jnp.pad(table, ((0, rows - acc), (0, 0)))

    h_tab = jnp.array(hs, dtype=jnp.float32)
    w_tab = jnp.array(ws, dtype=jnp.float32)
    base_tab = jnp.array(bases, dtype=jnp.int32)
    fh = jnp.take(h_tab, lev)
    fw = jnp.take(w_tab, lev)
    t = jnp.arange(_CROP, dtype=jnp.float32) / float(_CROP - 1)
    ys = jnp.zeros((k, _CROP), jnp.float32)
    xs = jnp.zeros((k, _CROP), jnp.float32)
    for i in range(len(fpn)):
        fhi = float(hs[i])
        fwi = float(ws[i])
        y1b = bs[:, 1] / imgf[1] * fhi / (fhi - 1.0)
        x1b = bs[:, 0] / imgf[2] * fwi / (fwi - 1.0)
        y2b = (bs[:, 3] / imgf[1] * fhi - 1.0) / (fhi - 1.0)
        x2b = (bs[:, 2] / imgf[2] * fwi - 1.0) / (fwi - 1.0)
        ys_i = y1b[:, None] * (fhi - 1.0) + t[None, :] * ((y2b - y1b) * (fhi - 1.0))[:, None]
        xs_i = x1b[:, None] * (fwi - 1.0) + t[None, :] * ((x2b - x1b) * (fwi - 1.0))[:, None]
        m = (lev == i)[:, None]
        ys = jnp.where(m, ys_i, ys)
        xs = jnp.where(m, xs_i, xs)
    wp_tab = jnp.array(wps, dtype=jnp.int32)
    pi = jnp.stack([
        fh.astype(jnp.int32), fw.astype(jnp.int32),
        jnp.take(wp_tab, lev), jnp.take(base_tab, lev),
    ], axis=1)
    xs_t = xs.reshape(k, _CROP, 1)

    grid_spec = pltpu.PrefetchScalarGridSpec(
        num_scalar_prefetch=3,
        grid=(k,),
        in_specs=[
            pl.BlockSpec((n, 4), lambda j, *_: (0, 0)),
            pl.BlockSpec((n, nc), lambda j, *_: (0, 0)),
            pl.BlockSpec((rows, c), lambda j, *_: (0, 0)),
            pl.BlockSpec((1, _CROP, 1), lambda j, *_: (j, 0, 0)),
        ],
        out_specs=[
            pl.BlockSpec((1, 1, 4), lambda j, *_: (j, 0, 0)),
            pl.BlockSpec((1, 1, nc), lambda j, *_: (j, 0, 0)),
            pl.BlockSpec((1, _CROP, _CROP, c), lambda j, *_: (j, 0, 0, 0)),
        ],
    )
    boxes_out, cls_out, rois = pl.pallas_call(
        _roi_kernel,
        grid_spec=grid_spec,
        out_shape=[
            jax.ShapeDtypeStruct((k, 1, 4), jnp.float32),
            jax.ShapeDtypeStruct((k, 1, nc), jnp.float32),
            jax.ShapeDtypeStruct((k, _CROP, _CROP, c), jnp.float32),
        ],
        compiler_params=pltpu.CompilerParams(
            dimension_semantics=("arbitrary",),
        ),
    )(sel, ys, pi, b_all, cls_all, table, xs_t)

    return (boxes_out.reshape(k, 4)[None],
            cls_out.reshape(k, nc)[None],
            rois[None])


# speedup vs baseline: 29.7904x; 1.8325x over previous
import functools

import jax
import jax.numpy as jnp
from jax.experimental import pallas as pl
from jax.experimental.pallas import tpu as pltpu

_CROP = 14
_K = 500
_WMAX = 128


def _scores_kernel(cls_ref, out_ref):
    out_ref[...] = jnp.max(cls_ref[...], axis=1, keepdims=True)


def _roi_kernel(sel_ref, ys_ref, pi_ref, boxes_ref, cls_ref, table_ref, xs_ref,
                boxes_out_ref, cls_out_ref, rois_ref):
    j = pl.program_id(0)
    src = sel_ref[j]

    boxes_out_ref[...] = boxes_ref[pl.ds(src, 1), :][None]
    cls_out_ref[...] = cls_ref[pl.ds(src, 1), :][None]

    h_i = pi_ref[j, 0]
    w_i = pi_ref[j, 1]
    wp_i = pi_ref[j, 2]
    base = pi_ref[j, 3]
    hf = h_i.astype(jnp.float32)
    wf = w_i.astype(jnp.float32)

    col = jax.lax.broadcasted_iota(jnp.int32, (_CROP, _WMAX), 1).astype(jnp.float32)
    xs = xs_ref[0]
    x0f = jnp.floor(xs)
    wx = xs - x0f
    x0 = jnp.clip(x0f, 0.0, wf - 1.0)
    xp = jnp.clip(x0f + 1.0, 0.0, wf - 1.0)
    vx = ((xs >= 0.0) & (xs <= wf - 1.0)).astype(jnp.float32)
    cmat = ((col == x0).astype(jnp.float32) * (1.0 - wx)
            + (col == xp).astype(jnp.float32) * wx) * vx

    for iy in range(_CROP):
        ysv = ys_ref[j, iy]
        y0f = jnp.floor(ysv)
        wy = ysv - y0f
        y0 = jnp.clip(y0f, 0.0, hf - 1.0).astype(jnp.int32)
        yp = jnp.clip(y0f + 1.0, 0.0, hf - 1.0).astype(jnp.int32)
        vy = ((ysv >= 0.0) & (ysv <= hf - 1.0)).astype(jnp.float32)
        s0 = pl.multiple_of(base + y0 * wp_i, 8)
        s1 = pl.multiple_of(base + yp * wp_i, 8)
        r0 = table_ref[pl.ds(s0, _WMAX), :]
        r1 = table_ref[pl.ds(s1, _WMAX), :]
        out_iy = (jnp.dot(cmat * ((1.0 - wy) * vy), r0,
                          preferred_element_type=jnp.float32)
                  + jnp.dot(cmat * (wy * vy), r1,
                            preferred_element_type=jnp.float32))
        rois_ref[0, iy, :, :] = out_iy


def kernel(image_shape, boxes, classification, fpn0, fpn1, fpn2, fpn3, fpn4):
    fpn = [fpn0[0], fpn1[0], fpn2[0], fpn3[0], fpn4[0]]
    b_all = boxes[0]
    cls_all = classification[0]
    n = b_all.shape[0]
    nc = cls_all.shape[1]
    c = fpn[0].shape[-1]
    imgf = image_shape.astype(jnp.float32)

    scores = pl.pallas_call(
        _scores_kernel,
        out_shape=jax.ShapeDtypeStruct((n, 1), jnp.float32),
    )(cls_all)[:, 0]

    k = min(_K, n)
    _, idx = jax.lax.top_k(scores, k)
    bsel = jnp.take(b_all, idx, axis=0)

    bw = bsel[:, 2] - bsel[:, 0]
    bh = bsel[:, 3] - bsel[:, 1]
    size = jnp.sqrt(bw * bh)
    levels = jnp.floor(1.0 + jnp.log2(size / 224.0 + 1e-7))
    levels = jnp.clip(levels, 0.0, 4.0).astype(jnp.int32)
    order = jnp.argsort(levels, stable=True)
    sel = jnp.take(idx, order, axis=0)
    lev = jnp.take(levels, order, axis=0)
    bs = jnp.take(bsel, order, axis=0)

    hs = [f.shape[0] for f in fpn]
    ws = [f.shape[1] for f in fpn]
    wps = [ww + (-ww) % 8 for ww in ws]
    bases = []
    acc = 0
    for hh, wp in zip(hs, wps):
        bases.append(acc)
        acc += hh * wp
    rows = acc + _WMAX
    rows += (-rows) % 8
    table = jnp.concatenate(
        [jnp.pad(f, ((0, 0), (0, wp - f.shape[1]), (0, 0))).reshape(-1, c)
         for f, wp in zip(fpn, wps)], axis=0)
    table = jnp.pad(table, ((0, rows - acc), (0, 0)))

    h_tab = jnp.array(hs, dtype=jnp.float32)
    w_tab = jnp.array(ws, dtype=jnp.float32)
    base_tab = jnp.array(bases, dtype=jnp.int32)
    fh = jnp.take(h_tab, lev)
    fw = jnp.take(w_tab, lev)
    t = jnp.arange(_CROP, dtype=jnp.float32) / float(_CROP - 1)
    ys = jnp.zeros((k, _CROP), jnp.float32)
    xs = jnp.zeros((k, _CROP), jnp.float32)
    for i in range(len(fpn)):
        fhi = float(hs[i])
        fwi = float(ws[i])
        y1b = bs[:, 1] / imgf[1] * fhi / (fhi - 1.0)
        x1b = bs[:, 0] / imgf[2] * fwi / (fwi - 1.0)
        y2b = (bs[:, 3] / imgf[1] * fhi - 1.0) / (fhi - 1.0)
        x2b = (bs[:, 2] / imgf[2] * fwi - 1.0) / (fwi - 1.0)
        ys_i = y1b[:, None] * (fhi - 1.0) + t[None, :] * ((y2b - y1b) * (fhi - 1.0))[:, None]
        xs_i = x1b[:, None] * (fwi - 1.0) + t[None, :] * ((x2b - x1b) * (fwi - 1.0))[:, None]
        m = (lev == i)[:, None]
        ys = jnp.where(m, ys_i, ys)
        xs = jnp.where(m, xs_i, xs)
    wp_tab = jnp.array(wps, dtype=jnp.int32)
    pi = jnp.stack([
        fh.astype(jnp.int32), fw.astype(jnp.int32),
        jnp.take(wp_tab, lev), jnp.take(base_tab, lev),
    ], axis=1)
    xs_t = xs.reshape(k, _CROP, 1)

    grid_spec = pltpu.PrefetchScalarGridSpec(
        num_scalar_prefetch=3,
        grid=(k,),
        in_specs=[
            pl.BlockSpec((n, 4), lambda j, *_: (0, 0)),
            pl.BlockSpec((n, nc), lambda j, *_: (0, 0)),
            pl.BlockSpec((rows, c), lambda j, *_: (0, 0)),
            pl.BlockSpec((1, _CROP, 1), lambda j, *_: (j, 0, 0)),
        ],
        out_specs=[
            pl.BlockSpec((1, 1, 4), lambda j, *_: (j, 0, 0)),
            pl.BlockSpec((1, 1, nc), lambda j, *_: (j, 0, 0)),
            pl.BlockSpec((1, _CROP, _CROP, c), lambda j, *_: (j, 0, 0, 0)),
        ],
    )
    boxes_out, cls_out, rois = pl.pallas_call(
        _roi_kernel,
        grid_spec=grid_spec,
        out_shape=[
            jax.ShapeDtypeStruct((k, 1, 4), jnp.float32),
            jax.ShapeDtypeStruct((k, 1, nc), jnp.float32),
            jax.ShapeDtypeStruct((k, _CROP, _CROP, c), jnp.float32),
        ],
        compiler_params=pltpu.CompilerParams(
            dimension_semantics=("arbitrary",),
        ),
    )(sel, ys, pi, b_all, cls_all, table, xs_t)

    return (boxes_out.reshape(k, 4)[None],
            cls_out.reshape(k, nc)[None],
            rois[None])


# speedup vs baseline: 30.0265x; 1.0079x over previous
import functools

import jax
import jax.numpy as jnp
from jax.experimental import pallas as pl
from jax.experimental.pallas import tpu as pltpu

_CROP = 14
_K = 500
_WMAX = 128


def _make_prep_kernel(shapes, wps, bases, rows, c):
    def prep_kernel(cls_ref, f0, f1, f2, f3, f4, scores_ref, table_ref):
        scores_ref[...] = jnp.max(cls_ref[...], axis=1, keepdims=True)
        frefs = [f0, f1, f2, f3, f4]
        for (hh, ww), wp, base, fref in zip(shapes, wps, bases, frefs):
            for r in range(hh):
                s = base + r * wp
                table_ref[s:s + ww, :] = fref[0, r, :, :]
                if wp > ww:
                    table_ref[s + ww:s + wp, :] = jnp.zeros((wp - ww, c),
                                                            jnp.float32)
        acc = bases[-1] + shapes[-1][0] * wps[-1]
        table_ref[acc:rows, :] = jnp.zeros((rows - acc, c), jnp.float32)
    return prep_kernel


def _roi_kernel(sel_ref, ys_ref, pi_ref, boxes_ref, cls_ref, table_ref, xs_ref,
                boxes_out_ref, cls_out_ref, rois_ref):
    j = pl.program_id(0)
    src = sel_ref[j]

    boxes_out_ref[...] = boxes_ref[pl.ds(src, 1), :][None]
    cls_out_ref[...] = cls_ref[pl.ds(src, 1), :][None]

    h_i = pi_ref[j, 0]
    w_i = pi_ref[j, 1]
    wp_i = pi_ref[j, 2]
    base = pi_ref[j, 3]
    hf = h_i.astype(jnp.float32)
    wf = w_i.astype(jnp.float32)

    col = jax.lax.broadcasted_iota(jnp.int32, (_CROP, _WMAX), 1).astype(jnp.float32)
    xs = xs_ref[0]
    x0f = jnp.floor(xs)
    wx = xs - x0f
    x0 = jnp.clip(x0f, 0.0, wf - 1.0)
    xp = jnp.clip(x0f + 1.0, 0.0, wf - 1.0)
    vx = ((xs >= 0.0) & (xs <= wf - 1.0)).astype(jnp.float32)
    cmat = ((col == x0).astype(jnp.float32) * (1.0 - wx)
            + (col == xp).astype(jnp.float32) * wx) * vx

    for iy in range(_CROP):
        ysv = ys_ref[j, iy]
        y0f = jnp.floor(ysv)
        wy = ysv - y0f
        y0 = jnp.clip(y0f, 0.0, hf - 1.0).astype(jnp.int32)
        yp = jnp.clip(y0f + 1.0, 0.0, hf - 1.0).astype(jnp.int32)
        vy = ((ysv >= 0.0) & (ysv <= hf - 1.0)).astype(jnp.float32)
        s0 = pl.multiple_of(base + y0 * wp_i, 8)
        s1 = pl.multiple_of(base + yp * wp_i, 8)
        r0 = table_ref[pl.ds(s0, _WMAX), :]
        r1 = table_ref[pl.ds(s1, _WMAX), :]
        out_iy = (jnp.dot(cmat * ((1.0 - wy) * vy), r0,
                          preferred_element_type=jnp.float32)
                  + jnp.dot(cmat * (wy * vy), r1,
                            preferred_element_type=jnp.float32))
        rois_ref[0, iy, :, :] = out_iy


def kernel(image_shape, boxes, classification, fpn0, fpn1, fpn2, fpn3, fpn4):
    fpn = [fpn0[0], fpn1[0], fpn2[0], fpn3[0], fpn4[0]]
    b_all = boxes[0]
    cls_all = classification[0]
    n = b_all.shape[0]
    nc = cls_all.shape[1]
    c = fpn[0].shape[-1]
    imgf = image_shape.astype(jnp.float32)

    hs = [f.shape[0] for f in fpn]
    ws = [f.shape[1] for f in fpn]
    wps = [ww + (-ww) % 8 for ww in ws]
    bases = []
    acc = 0
    for hh, wp in zip(hs, wps):
        bases.append(acc)
        acc += hh * wp
    rows = acc + _WMAX
    rows += (-rows) % 8
    shapes = list(zip(hs, ws))
    scores, table = pl.pallas_call(
        _make_prep_kernel(shapes, wps, bases, rows, c),
        out_shape=[
            jax.ShapeDtypeStruct((n, 1), jnp.float32),
            jax.ShapeDtypeStruct((rows, c), jnp.float32),
        ],
    )(cls_all, fpn0, fpn1, fpn2, fpn3, fpn4)
    scores = scores[:, 0]

    k = min(_K, n)
    _, idx = jax.lax.top_k(scores, k)
    bsel = jnp.take(b_all, idx, axis=0)

    bw = bsel[:, 2] - bsel[:, 0]
    bh = bsel[:, 3] - bsel[:, 1]
    size = jnp.sqrt(bw * bh)
    levels = jnp.floor(1.0 + jnp.log2(size / 224.0 + 1e-7))
    levels = jnp.clip(levels, 0.0, 4.0).astype(jnp.int32)
    order = jnp.argsort(levels, stable=True)
    sel = jnp.take(idx, order, axis=0)
    lev = jnp.take(levels, order, axis=0)
    bs = jnp.take(bsel, order, axis=0)

    h_tab = jnp.array(hs, dtype=jnp.float32)
    w_tab = jnp.array(ws, dtype=jnp.float32)
    base_tab = jnp.array(bases, dtype=jnp.int32)
    fh = jnp.take(h_tab, lev)
    fw = jnp.take(w_tab, lev)
    t = jnp.arange(_CROP, dtype=jnp.float32) / float(_CROP - 1)
    ys = jnp.zeros((k, _CROP), jnp.float32)
    xs = jnp.zeros((k, _CROP), jnp.float32)
    for i in range(len(fpn)):
        fhi = float(hs[i])
        fwi = float(ws[i])
        y1b = bs[:, 1] / imgf[1] * fhi / (fhi - 1.0)
        x1b = bs[:, 0] / imgf[2] * fwi / (fwi - 1.0)
        y2b = (bs[:, 3] / imgf[1] * fhi - 1.0) / (fhi - 1.0)
        x2b = (bs[:, 2] / imgf[2] * fwi - 1.0) / (fwi - 1.0)
        ys_i = y1b[:, None] * (fhi - 1.0) + t[None, :] * ((y2b - y1b) * (fhi - 1.0))[:, None]
        xs_i = x1b[:, None] * (fwi - 1.0) + t[None, :] * ((x2b - x1b) * (fwi - 1.0))[:, None]
        m = (lev == i)[:, None]
        ys = jnp.where(m, ys_i, ys)
        xs = jnp.where(m, xs_i, xs)
    wp_tab = jnp.array(wps, dtype=jnp.int32)
    pi = jnp.stack([
        fh.astype(jnp.int32), fw.astype(jnp.int32),
        jnp.take(wp_tab, lev), jnp.take(base_tab, lev),
    ], axis=1)
    xs_t = xs.reshape(k, _CROP, 1)

    grid_spec = pltpu.PrefetchScalarGridSpec(
        num_scalar_prefetch=3,
        grid=(k,),
        in_specs=[
            pl.BlockSpec((n, 4), lambda j, *_: (0, 0)),
            pl.BlockSpec((n, nc), lambda j, *_: (0, 0)),
            pl.BlockSpec((rows, c), lambda j, *_: (0, 0)),
            pl.BlockSpec((1, _CROP, 1), lambda j, *_: (j, 0, 0)),
        ],
        out_specs=[
            pl.BlockSpec((1, 1, 4), lambda j, *_: (j, 0, 0)),
            pl.BlockSpec((1, 1, nc), lambda j, *_: (j, 0, 0)),
            pl.BlockSpec((1, _CROP, _CROP, c), lambda j, *_: (j, 0, 0, 0)),
        ],
    )
    boxes_out, cls_out, rois = pl.pallas_call(
        _roi_kernel,
        grid_spec=grid_spec,
        out_shape=[
            jax.ShapeDtypeStruct((k, 1, 4), jnp.float32),
            jax.ShapeDtypeStruct((k, 1, nc), jnp.float32),
            jax.ShapeDtypeStruct((k, _CROP, _CROP, c), jnp.float32),
        ],
        compiler_params=pltpu.CompilerParams(
            dimension_semantics=("arbitrary",),
        ),
    )(sel, ys, pi, b_all, cls_all, table, xs_t)

    return (boxes_out.reshape(k, 4)[None],
            cls_out.reshape(k, nc)[None],
            rois[None])


# speedup vs baseline: 43.8184x; 1.4593x over previous
import functools

import jax
import jax.numpy as jnp
from jax.experimental import pallas as pl
from jax.experimental.pallas import tpu as pltpu

_CROP = 14
_K = 500
_WMAX = 128


def _make_prep_kernel(shapes, wps, bases, rows, c):
    def prep_kernel(cls_ref, f0, f1, f2, f3, f4, scores_ref, table_ref):
        scores_ref[...] = jnp.max(cls_ref[...], axis=1, keepdims=True)
        frefs = [f0, f1, f2, f3, f4]
        for (hh, ww), wp, base, fref in zip(shapes, wps, bases, frefs):
            for r in range(hh):
                s = base + r * wp
                table_ref[s:s + ww, :] = fref[0, r, :, :]
                if wp > ww:
                    table_ref[s + ww:s + wp, :] = jnp.zeros((wp - ww, c),
                                                            jnp.float32)
        acc = bases[-1] + shapes[-1][0] * wps[-1]
        table_ref[acc:rows, :] = jnp.zeros((rows - acc, c), jnp.float32)
    return prep_kernel


def _roi_kernel(sel_ref, ys_ref, pi_ref, boxes_ref, cls_ref, table_ref, xs_ref,
                boxes_out_ref, cls_out_ref, rois_ref):
    j = pl.program_id(0)
    src = sel_ref[j]

    boxes_out_ref[...] = boxes_ref[pl.ds(src, 1), :][None]
    cls_out_ref[...] = cls_ref[pl.ds(src, 1), :][None]

    h_i = pi_ref[j, 0]
    w_i = pi_ref[j, 1]
    wp_i = pi_ref[j, 2]
    base = pi_ref[j, 3]
    hf = h_i.astype(jnp.float32)
    wf = w_i.astype(jnp.float32)

    col = jax.lax.broadcasted_iota(jnp.int32, (_CROP, _WMAX), 1).astype(jnp.float32)
    xs = xs_ref[0]
    x0f = jnp.floor(xs)
    wx = xs - x0f
    x0 = jnp.clip(x0f, 0.0, wf - 1.0)
    xp = jnp.clip(x0f + 1.0, 0.0, wf - 1.0)
    vx = ((xs >= 0.0) & (xs <= wf - 1.0)).astype(jnp.float32)
    cmat = ((col == x0).astype(jnp.float32) * (1.0 - wx)
            + (col == xp).astype(jnp.float32) * wx) * vx

    for iy in range(_CROP):
        ysv = ys_ref[j, iy]
        y0f = jnp.floor(ysv)
        wy = ysv - y0f
        y0 = jnp.clip(y0f, 0.0, hf - 1.0).astype(jnp.int32)
        yp = jnp.clip(y0f + 1.0, 0.0, hf - 1.0).astype(jnp.int32)
        vy = ((ysv >= 0.0) & (ysv <= hf - 1.0)).astype(jnp.float32)
        s0 = pl.multiple_of(base + y0 * wp_i, 8)
        s1 = pl.multiple_of(base + yp * wp_i, 8)
        r0 = table_ref[pl.ds(s0, _WMAX), :]
        r1 = table_ref[pl.ds(s1, _WMAX), :]
        out_iy = (jnp.dot(cmat * ((1.0 - wy) * vy), r0,
                          preferred_element_type=jnp.float32)
                  + jnp.dot(cmat * (wy * vy), r1,
                            preferred_element_type=jnp.float32))
        rois_ref[0, 0, iy, :, :] = out_iy


def kernel(image_shape, boxes, classification, fpn0, fpn1, fpn2, fpn3, fpn4):
    fpn = [fpn0[0], fpn1[0], fpn2[0], fpn3[0], fpn4[0]]
    b_all = boxes[0]
    cls_all = classification[0]
    n = b_all.shape[0]
    nc = cls_all.shape[1]
    c = fpn[0].shape[-1]
    imgf = image_shape.astype(jnp.float32)

    hs = [f.shape[0] for f in fpn]
    ws = [f.shape[1] for f in fpn]
    wps = [ww + (-ww) % 8 for ww in ws]
    bases = []
    acc = 0
    for hh, wp in zip(hs, wps):
        bases.append(acc)
        acc += hh * wp
    rows = acc + _WMAX
    rows += (-rows) % 8
    shapes = list(zip(hs, ws))
    scores, table = pl.pallas_call(
        _make_prep_kernel(shapes, wps, bases, rows, c),
        out_shape=[
            jax.ShapeDtypeStruct((n, 1), jnp.float32),
            jax.ShapeDtypeStruct((rows, c), jnp.float32),
        ],
    )(cls_all, fpn0, fpn1, fpn2, fpn3, fpn4)
    scores = scores[:, 0]

    k = min(_K, n)
    _, idx = jax.lax.top_k(scores, k)
    bsel = jnp.take(b_all, idx, axis=0)

    bw = bsel[:, 2] - bsel[:, 0]
    bh = bsel[:, 3] - bsel[:, 1]
    size = jnp.sqrt(bw * bh)
    levels = jnp.floor(1.0 + jnp.log2(size / 224.0 + 1e-7))
    levels = jnp.clip(levels, 0.0, 4.0).astype(jnp.int32)
    order = jnp.argsort(levels, stable=True)
    sel = jnp.take(idx, order, axis=0)
    lev = jnp.take(levels, order, axis=0)
    bs = jnp.take(bsel, order, axis=0)

    h_tab = jnp.array(hs, dtype=jnp.float32)
    w_tab = jnp.array(ws, dtype=jnp.float32)
    base_tab = jnp.array(bases, dtype=jnp.int32)
    fh = jnp.take(h_tab, lev)
    fw = jnp.take(w_tab, lev)
    t = jnp.arange(_CROP, dtype=jnp.float32) / float(_CROP - 1)
    ys = jnp.zeros((k, _CROP), jnp.float32)
    xs = jnp.zeros((k, _CROP), jnp.float32)
    for i in range(len(fpn)):
        fhi = float(hs[i])
        fwi = float(ws[i])
        y1b = bs[:, 1] / imgf[1] * fhi / (fhi - 1.0)
        x1b = bs[:, 0] / imgf[2] * fwi / (fwi - 1.0)
        y2b = (bs[:, 3] / imgf[1] * fhi - 1.0) / (fhi - 1.0)
        x2b = (bs[:, 2] / imgf[2] * fwi - 1.0) / (fwi - 1.0)
        ys_i = y1b[:, None] * (fhi - 1.0) + t[None, :] * ((y2b - y1b) * (fhi - 1.0))[:, None]
        xs_i = x1b[:, None] * (fwi - 1.0) + t[None, :] * ((x2b - x1b) * (fwi - 1.0))[:, None]
        m = (lev == i)[:, None]
        ys = jnp.where(m, ys_i, ys)
        xs = jnp.where(m, xs_i, xs)
    wp_tab = jnp.array(wps, dtype=jnp.int32)
    pi = jnp.stack([
        fh.astype(jnp.int32), fw.astype(jnp.int32),
        jnp.take(wp_tab, lev), jnp.take(base_tab, lev),
    ], axis=1)
    xs_t = xs.reshape(k, _CROP, 1)

    grid_spec = pltpu.PrefetchScalarGridSpec(
        num_scalar_prefetch=3,
        grid=(k,),
        in_specs=[
            pl.BlockSpec((n, 4), lambda j, *_: (0, 0)),
            pl.BlockSpec((n, nc), lambda j, *_: (0, 0)),
            pl.BlockSpec((rows, c), lambda j, *_: (0, 0)),
            pl.BlockSpec((1, _CROP, 1), lambda j, *_: (j, 0, 0)),
        ],
        out_specs=[
            pl.BlockSpec((1, 1, 4), lambda j, *_: (j, 0, 0)),
            pl.BlockSpec((1, 1, nc), lambda j, *_: (j, 0, 0)),
            pl.BlockSpec((1, 1, _CROP, _CROP, c), lambda j, *_: (0, j, 0, 0, 0)),
        ],
    )
    boxes_out, cls_out, rois = pl.pallas_call(
        _roi_kernel,
        grid_spec=grid_spec,
        out_shape=[
            jax.ShapeDtypeStruct((k, 1, 4), jnp.float32),
            jax.ShapeDtypeStruct((k, 1, nc), jnp.float32),
            jax.ShapeDtypeStruct((1, k, _CROP, _CROP, c), jnp.float32),
        ],
        compiler_params=pltpu.CompilerParams(
            dimension_semantics=("arbitrary",),
        ),
    )(sel, ys, pi, b_all, cls_all, table, xs_t)

    return (boxes_out.reshape(k, 4)[None],
            cls_out.reshape(k, nc)[None],
            rois)


# speedup vs baseline: 44.2451x; 1.0097x over previous
import functools

import jax
import jax.numpy as jnp
from jax.experimental import pallas as pl
from jax.experimental.pallas import tpu as pltpu

_CROP = 14
_K = 500
_WMAX = 104


def _make_prep_kernel(shapes, wps, bases, rows, c):
    def prep_kernel(cls_ref, f0, f1, f2, f3, f4, scores_ref, table_ref):
        scores_ref[...] = jnp.max(cls_ref[...], axis=1, keepdims=True)
        frefs = [f0, f1, f2, f3, f4]
        for (hh, ww), wp, base, fref in zip(shapes, wps, bases, frefs):
            for r in range(hh):
                s = base + r * wp
                table_ref[s:s + ww, :] = fref[0, r, :, :]
                if wp > ww:
                    table_ref[s + ww:s + wp, :] = jnp.zeros((wp - ww, c),
                                                            jnp.float32)
        acc = bases[-1] + shapes[-1][0] * wps[-1]
        table_ref[acc:rows, :] = jnp.zeros((rows - acc, c), jnp.float32)
    return prep_kernel


def _roi_kernel(sel_ref, ys_ref, pi_ref, boxes_ref, cls_ref, table_ref, xs_ref,
                boxes_out_ref, cls_out_ref, rois_ref):
    j = pl.program_id(0)
    src = sel_ref[j]

    boxes_out_ref[...] = boxes_ref[pl.ds(src, 1), :][None]
    cls_out_ref[...] = cls_ref[pl.ds(src, 1), :][None]

    h_i = pi_ref[j, 0]
    w_i = pi_ref[j, 1]
    wp_i = pi_ref[j, 2]
    base = pi_ref[j, 3]
    hf = h_i.astype(jnp.float32)
    wf = w_i.astype(jnp.float32)

    col = jax.lax.broadcasted_iota(jnp.int32, (_CROP, _WMAX), 1).astype(jnp.float32)
    xs = xs_ref[0]
    x0f = jnp.floor(xs)
    wx = xs - x0f
    x0 = jnp.clip(x0f, 0.0, wf - 1.0)
    xp = jnp.clip(x0f + 1.0, 0.0, wf - 1.0)
    vx = ((xs >= 0.0) & (xs <= wf - 1.0)).astype(jnp.float32)
    cmat = ((col == x0).astype(jnp.float32) * (1.0 - wx)
            + (col == xp).astype(jnp.float32) * wx) * vx

    for iy in range(_CROP):
        ysv = ys_ref[j, iy]
        y0f = jnp.floor(ysv)
        wy = ysv - y0f
        y0 = jnp.clip(y0f, 0.0, hf - 1.0).astype(jnp.int32)
        yp = jnp.clip(y0f + 1.0, 0.0, hf - 1.0).astype(jnp.int32)
        vy = ((ysv >= 0.0) & (ysv <= hf - 1.0)).astype(jnp.float32)
        s0 = pl.multiple_of(base + y0 * wp_i, 8)
        s1 = pl.multiple_of(base + yp * wp_i, 8)
        r0 = table_ref[pl.ds(s0, _WMAX), :]
        r1 = table_ref[pl.ds(s1, _WMAX), :]
        out_iy = (jnp.dot(cmat * ((1.0 - wy) * vy), r0,
                          preferred_element_type=jnp.float32)
                  + jnp.dot(cmat * (wy * vy), r1,
                            preferred_element_type=jnp.float32))
        rois_ref[0, 0, iy, :, :] = out_iy


def kernel(image_shape, boxes, classification, fpn0, fpn1, fpn2, fpn3, fpn4):
    fpn = [fpn0[0], fpn1[0], fpn2[0], fpn3[0], fpn4[0]]
    b_all = boxes[0]
    cls_all = classification[0]
    n = b_all.shape[0]
    nc = cls_all.shape[1]
    c = fpn[0].shape[-1]
    imgf = image_shape.astype(jnp.float32)

    hs = [f.shape[0] for f in fpn]
    ws = [f.shape[1] for f in fpn]
    wps = [ww + (-ww) % 8 for ww in ws]
    bases = []
    acc = 0
    for hh, wp in zip(hs, wps):
        bases.append(acc)
        acc += hh * wp
    rows = acc + _WMAX
    rows += (-rows) % 8
    shapes = list(zip(hs, ws))
    scores, table = pl.pallas_call(
        _make_prep_kernel(shapes, wps, bases, rows, c),
        out_shape=[
            jax.ShapeDtypeStruct((n, 1), jnp.float32),
            jax.ShapeDtypeStruct((rows, c), jnp.float32),
        ],
    )(cls_all, fpn0, fpn1, fpn2, fpn3, fpn4)
    scores = scores[:, 0]

    k = min(_K, n)
    _, idx = jax.lax.top_k(scores, k)
    bsel = jnp.take(b_all, idx, axis=0)

    bw = bsel[:, 2] - bsel[:, 0]
    bh = bsel[:, 3] - bsel[:, 1]
    size = jnp.sqrt(bw * bh)
    levels = jnp.floor(1.0 + jnp.log2(size / 224.0 + 1e-7))
    levels = jnp.clip(levels, 0.0, 4.0).astype(jnp.int32)
    order = jnp.argsort(levels, stable=True)
    sel = jnp.take(idx, order, axis=0)
    lev = jnp.take(levels, order, axis=0)
    bs = jnp.take(bsel, order, axis=0)

    h_tab = jnp.array(hs, dtype=jnp.float32)
    w_tab = jnp.array(ws, dtype=jnp.float32)
    base_tab = jnp.array(bases, dtype=jnp.int32)
    fh = jnp.take(h_tab, lev)
    fw = jnp.take(w_tab, lev)
    t = jnp.arange(_CROP, dtype=jnp.float32) / float(_CROP - 1)
    ys = jnp.zeros((k, _CROP), jnp.float32)
    xs = jnp.zeros((k, _CROP), jnp.float32)
    for i in range(len(fpn)):
        fhi = float(hs[i])
        fwi = float(ws[i])
        y1b = bs[:, 1] / imgf[1] * fhi / (fhi - 1.0)
        x1b = bs[:, 0] / imgf[2] * fwi / (fwi - 1.0)
        y2b = (bs[:, 3] / imgf[1] * fhi - 1.0) / (fhi - 1.0)
        x2b = (bs[:, 2] / imgf[2] * fwi - 1.0) / (fwi - 1.0)
        ys_i = y1b[:, None] * (fhi - 1.0) + t[None, :] * ((y2b - y1b) * (fhi - 1.0))[:, None]
        xs_i = x1b[:, None] * (fwi - 1.0) + t[None, :] * ((x2b - x1b) * (fwi - 1.0))[:, None]
        m = (lev == i)[:, None]
        ys = jnp.where(m, ys_i, ys)
        xs = jnp.where(m, xs_i, xs)
    wp_tab = jnp.array(wps, dtype=jnp.int32)
    pi = jnp.stack([
        fh.astype(jnp.int32), fw.astype(jnp.int32),
        jnp.take(wp_tab, lev), jnp.take(base_tab, lev),
    ], axis=1)
    xs_t = xs.reshape(k, _CROP, 1)

    grid_spec = pltpu.PrefetchScalarGridSpec(
        num_scalar_prefetch=3,
        grid=(k,),
        in_specs=[
            pl.BlockSpec((n, 4), lambda j, *_: (0, 0)),
            pl.BlockSpec((n, nc), lambda j, *_: (0, 0)),
            pl.BlockSpec((rows, c), lambda j, *_: (0, 0)),
            pl.BlockSpec((1, _CROP, 1), lambda j, *_: (j, 0, 0)),
        ],
        out_specs=[
            pl.BlockSpec((1, 1, 4), lambda j, *_: (j, 0, 0)),
            pl.BlockSpec((1, 1, nc), lambda j, *_: (j, 0, 0)),
            pl.BlockSpec((1, 1, _CROP, _CROP, c), lambda j, *_: (0, j, 0, 0, 0)),
        ],
    )
    boxes_out, cls_out, rois = pl.pallas_call(
        _roi_kernel,
        grid_spec=grid_spec,
        out_shape=[
            jax.ShapeDtypeStruct((k, 1, 4), jnp.float32),
            jax.ShapeDtypeStruct((k, 1, nc), jnp.float32),
            jax.ShapeDtypeStruct((1, k, _CROP, _CROP, c), jnp.float32),
        ],
        compiler_params=pltpu.CompilerParams(
            dimension_semantics=("arbitrary",),
        ),
    )(sel, ys, pi, b_all, cls_all, table, xs_t)

    return (boxes_out.reshape(k, 4)[None],
            cls_out.reshape(k, nc)[None],
            rois)
